# baseline (speedup 1.0000x reference)
import functools

import jax
import jax.numpy as jnp
from jax import lax
from jax.experimental import pallas as pl
from jax.experimental.pallas import tpu as pltpu
from jax.experimental.pallas import tpu_sc as plsc

B = 64
L = 50
BL = B * L
H = 200
HP = 256
G3 = 3 * HP
E = 100
EP = 128
BL_PAD = 3328


def _sc_gather(table, idx):
    info = plsc.get_sparse_core_info()
    nc, ns = info.num_cores, info.num_subcores
    nw = nc * ns
    b_per_w = BL_PAD // nw

    mesh = plsc.VectorSubcoreMesh(core_axis_name="c", subcore_axis_name="s")

    @functools.partial(
        pl.kernel,
        mesh=mesh,
        out_type=jax.ShapeDtypeStruct((BL_PAD, EP), jnp.float32),
        scratch_types=[
            pltpu.VMEM((b_per_w,), jnp.int32),
            pltpu.VMEM((b_per_w, EP), jnp.float32),
            pltpu.SemaphoreType.DMA,
        ],
    )
    def gather_kernel(table_hbm, idx_hbm, out_hbm, idx_v, rows_v, sem):
        wid = lax.axis_index("s") * nc + lax.axis_index("c")
        base = wid * b_per_w
        pltpu.sync_copy(idx_hbm.at[pl.ds(base, b_per_w)], idx_v)
        pltpu.async_copy(table_hbm.at[idx_v], rows_v, sem).wait()
        pltpu.sync_copy(rows_v, out_hbm.at[pl.ds(base, b_per_w)])

    return gather_kernel(table, idx)


def _gate_pad_rows(W):
    lead = W.shape[:-2]
    cin = W.shape[-1]
    Wg = W.reshape(lead + (3, H, cin))
    pad = [(0, 0)] * len(lead) + [(0, 0), (0, HP - H), (0, 0)]
    return jnp.pad(Wg, pad).reshape(lead + (G3, cin))


def _in_remap(W):
    z = jnp.zeros(W.shape[:-1] + (HP - H,), W.dtype)
    return jnp.concatenate([W[..., 0:H], z, W[..., H:2 * H], z], axis=-1)


def _nt(a, b):
    return lax.dot_general(a, b, (((1,), (1,)), ((), ())),
                           preferred_element_type=jnp.float32)


def _tc_body(x_ref, wih0, wih12, whh, bih, bhh,
             w1p, b1, w11, b11, w2p, b2,
             out_ref, gf, gb, y1, y2):
    xin = (x_ref, y1, y2)
    yout = (y1, y2, None)

    finals = []
    for l in range(3):
        wf = wih0[0] if l == 0 else wih12[2 * l - 2]
        wb = wih0[1] if l == 0 else wih12[2 * l - 1]
        bf = bih[2 * l:2 * l + 1, :]
        bb = bih[2 * l + 1:2 * l + 2, :]
        nch = 4
        rows = BL // nch
        for c in range(nch):
            xs = xin[l][c * rows:(c + 1) * rows, :]
            gf[c * rows:(c + 1) * rows, :] = _nt(xs, wf) + bf
            gb[c * rows:(c + 1) * rows, :] = _nt(xs, wb) + bb

        whf = whh[2 * l]
        whb = whh[2 * l + 1]
        bhf = bhh[2 * l:2 * l + 1, :]
        bhb = bhh[2 * l + 1:2 * l + 2, :]
        ydst = yout[l]

        def step(t, h, whf=whf, whb=whb, bhf=bhf, bhb=bhb, ydst=ydst):
            gif = gf[pl.ds(t * B, B), :]
            gib = gb[pl.ds((L - 1) * B - t * B, B), :]
            gi = jnp.concatenate([gif, gib], axis=0)
            hb16 = h.astype(jnp.bfloat16)
            ghf = lax.dot(hb16[0:B], whf, preferred_element_type=jnp.float32) + bhf
            ghb = lax.dot(hb16[B:2 * B], whb, preferred_element_type=jnp.float32) + bhb
            gh = jnp.concatenate([ghf, ghb], axis=0)
            rz = jax.nn.sigmoid(gi[:, 0:2 * HP] + gh[:, 0:2 * HP])
            r = rz[:, 0:HP]
            z = rz[:, HP:2 * HP]
            n = jnp.tanh(gi[:, 2 * HP:G3] + r * gh[:, 2 * HP:G3])
            hn = (1.0 - z) * n + z * h
            if ydst is not None:
                hnb = hn.astype(jnp.bfloat16)
                ydst[pl.ds(t * B, B), 0:HP] = hnb[0:B]
                ydst[pl.ds((L - 1) * B - t * B, B), HP:2 * HP] = hnb[B:2 * B]
            return hn

        h = lax.fori_loop(0, L, step, jnp.zeros((2 * B, HP), jnp.float32))
        finals.append(h[0:B])
        finals.append(h[B:2 * B])

    x1c = jnp.concatenate(finals, axis=1)
    h1 = jnp.maximum(_nt(x1c, w1p[...]) + b1[...], 0.0)
    h2 = jnp.maximum(_nt(h1, w11[...]) + b11[...], 0.0)
    out_ref[...] = jnp.maximum(_nt(h2, w2p[...]) + b2[...], 0.0)


def _tc_forward(x, args):
    return pl.pallas_call(
        _tc_body,
        out_shape=jax.ShapeDtypeStruct((B, 128), jnp.float32),
        scratch_shapes=[
            pltpu.VMEM((BL, G3), jnp.float32),
            pltpu.VMEM((BL, G3), jnp.float32),
            pltpu.VMEM((BL, 2 * HP), jnp.bfloat16),
            pltpu.VMEM((BL, 2 * HP), jnp.bfloat16),
        ],
    )(x, *args)


def kernel(dataTokens, embed,
           Wih0f, Whh0f, bih0f, bhh0f, Wih0b, Whh0b, bih0b, bhh0b,
           Wih1f, Whh1f, bih1f, bhh1f, Wih1b, Whh1b, bih1b, bhh1b,
           Wih2f, Whh2f, bih2f, bhh2f, Wih2b, Whh2b, bih2b, bhh2b,
           W1, b1, W11, b11, W2, b2):
    bft = jnp.bfloat16
    idx = dataTokens.T.reshape(-1).astype(jnp.int32)
    idx = jnp.pad(idx, (0, BL_PAD - BL))
    table = jnp.pad(embed, ((0, 0), (0, EP - E)))
    rows = _sc_gather(table, idx)
    x = rows[0:BL].astype(bft)

    wih0 = _gate_pad_rows(jnp.stack([Wih0f, Wih0b]))
    wih0 = jnp.pad(wih0, ((0, 0), (0, 0), (0, EP - E))).astype(bft)
    wih12 = _in_remap(_gate_pad_rows(
        jnp.stack([Wih1f, Wih1b, Wih2f, Wih2b]))).astype(bft)
    whh = _gate_pad_rows(jnp.stack(
        [Whh0f, Whh0b, Whh1f, Whh1b, Whh2f, Whh2b]))
    whh = jnp.pad(jnp.swapaxes(whh, 1, 2),
                  ((0, 0), (0, HP - H), (0, 0))).astype(bft)
    bih = jnp.pad(jnp.stack([bih0f, bih0b, bih1f, bih1b, bih2f, bih2b])
                  .reshape(6, 3, H), ((0, 0), (0, 0), (0, HP - H))).reshape(6, G3)
    bhh = jnp.pad(jnp.stack([bhh0f, bhh0b, bhh1f, bhh1b, bhh2f, bhh2b])
                  .reshape(6, 3, H), ((0, 0), (0, 0), (0, HP - H))).reshape(6, G3)

    w1p = jnp.pad(W1[:, 2000:].reshape(1000, 6, H),
                  ((0, 0), (0, 0), (0, HP - H))).reshape(1000, 6 * HP)
    w2p = jnp.pad(W2, ((0, 128 - 4), (0, 0)))
    b2p = jnp.pad(b2, (0, 128 - 4)).reshape(1, 128)

    out = _tc_forward(x, (wih0, wih12, whh, bih, bhh,
                          w1p, b1.reshape(1, 1000), W11, b11.reshape(1, 500),
                          w2p, b2p))
    return out[:, 0:4]


# speedup vs baseline: 1.2064x; 1.2064x over previous
import functools

import jax
import jax.numpy as jnp
from jax import lax
from jax.experimental import pallas as pl
from jax.experimental.pallas import tpu as pltpu
from jax.experimental.pallas import tpu_sc as plsc

B = 64
L = 50
BL = B * L
H = 200
HP = 256
G3 = 3 * HP
E = 100
EP = 128
BL_PAD = 3328


def _sc_gather(table, idx):
    info = plsc.get_sparse_core_info()
    nc, ns = info.num_cores, info.num_subcores
    nw = nc * ns
    b_per_w = BL_PAD // nw

    mesh = plsc.VectorSubcoreMesh(core_axis_name="c", subcore_axis_name="s")

    @functools.partial(
        pl.kernel,
        mesh=mesh,
        out_type=jax.ShapeDtypeStruct((BL_PAD, EP), jnp.float32),
        scratch_types=[
            pltpu.VMEM((b_per_w,), jnp.int32),
            pltpu.VMEM((b_per_w, EP), jnp.float32),
            pltpu.SemaphoreType.DMA,
        ],
    )
    def gather_kernel(table_hbm, idx_hbm, out_hbm, idx_v, rows_v, sem):
        wid = lax.axis_index("s") * nc + lax.axis_index("c")
        base = wid * b_per_w
        pltpu.sync_copy(idx_hbm.at[pl.ds(base, b_per_w)], idx_v)
        pltpu.async_copy(table_hbm.at[idx_v], rows_v, sem).wait()
        pltpu.sync_copy(rows_v, out_hbm.at[pl.ds(base, b_per_w)])

    return gather_kernel(table, idx)


def _gate_pad_rows(W):
    lead = W.shape[:-2]
    cin = W.shape[-1]
    Wg = W.reshape(lead + (3, H, cin))
    pad = [(0, 0)] * len(lead) + [(0, 0), (0, HP - H), (0, 0)]
    return jnp.pad(Wg, pad).reshape(lead + (G3, cin))


def _in_remap(W):
    z = jnp.zeros(W.shape[:-1] + (HP - H,), W.dtype)
    return jnp.concatenate([W[..., 0:H], z, W[..., H:2 * H], z], axis=-1)


def _nt(a, b):
    return lax.dot_general(a, b, (((1,), (1,)), ((), ())),
                           preferred_element_type=jnp.float32)


def _tc_body(x_ref, wih0, wih12, whh, bih, bhh,
             w1a, w1b, b1, w11, b11, w2p, b2,
             out_ref, gf, gb, y1, y2):
    xin = (x_ref, y1, y2)
    yout = (y1, y2, None)

    finals = []
    for l in range(3):
        wf = wih0[0] if l == 0 else wih12[2 * l - 2]
        wb = wih0[1] if l == 0 else wih12[2 * l - 1]
        bf = bih[2 * l:2 * l + 1, :]
        bb = bih[2 * l + 1:2 * l + 2, :]
        nch = 4
        rows = BL // nch
        for c in range(nch):
            if l == 0:
                xs = x_ref[c * rows:(c + 1) * rows, :].astype(jnp.bfloat16)
            else:
                xs = xin[l][c * rows:(c + 1) * rows, :]
            gf[c * rows:(c + 1) * rows, :] = (_nt(xs, wf) + bf).astype(jnp.bfloat16)
            gb[c * rows:(c + 1) * rows, :] = (_nt(xs, wb) + bb).astype(jnp.bfloat16)

        whf = whh[2 * l]
        whb = whh[2 * l + 1]
        bhf = bhh[2 * l:2 * l + 1, :]
        bhb = bhh[2 * l + 1:2 * l + 2, :]
        ydst = yout[l]

        def step(t, h, whf=whf, whb=whb, bhf=bhf, bhb=bhb, ydst=ydst):
            gif = gf[pl.ds(t * B, B), :]
            gib = gb[pl.ds((L - 1) * B - t * B, B), :]
            gi = jnp.concatenate([gif, gib], axis=0).astype(jnp.float32)
            hb16 = h.astype(jnp.bfloat16)
            ghf = lax.dot(hb16[0:B], whf, preferred_element_type=jnp.float32) + bhf
            ghb = lax.dot(hb16[B:2 * B], whb, preferred_element_type=jnp.float32) + bhb
            gh = jnp.concatenate([ghf, ghb], axis=0)
            rz = jax.nn.sigmoid(gi[:, 0:2 * HP] + gh[:, 0:2 * HP])
            r = rz[:, 0:HP]
            z = rz[:, HP:2 * HP]
            n = jnp.tanh(gi[:, 2 * HP:G3] + r * gh[:, 2 * HP:G3])
            hn = (1.0 - z) * n + z * h
            if ydst is not None:
                hnb = hn.astype(jnp.bfloat16)
                ydst[pl.ds(t * B, B), 0:HP] = hnb[0:B]
                ydst[pl.ds((L - 1) * B - t * B, B), HP:2 * HP] = hnb[B:2 * B]
            return hn

        h = lax.fori_loop(0, L, step, jnp.zeros((2 * B, HP), jnp.float32))
        finals.append(h[0:B])
        finals.append(h[B:2 * B])

    x1c = jnp.concatenate([f[:, 0:H] for f in finals], axis=1)
    h1 = jnp.maximum(_nt(x1c[:, 0:560], w1a[:, 80:640])
                     + _nt(x1c[:, 560:1200], w1b[...]) + b1[...], 0.0)
    h2 = jnp.maximum(_nt(h1, w11[...]) + b11[...], 0.0)
    out_ref[...] = jnp.maximum(_nt(h2, w2p[...]) + b2[...], 0.0)


def _tc_forward(x, args):
    full = pl.BlockSpec()
    in_specs = [full] * 6 + [pl.BlockSpec((1000, 640), lambda i: (0, 3)),
                             pl.BlockSpec((1000, 640), lambda i: (0, 4))] + [full] * 5
    return pl.pallas_call(
        _tc_body,
        out_shape=jax.ShapeDtypeStruct((B, 128), jnp.float32),
        grid=(1,),
        in_specs=in_specs,
        scratch_shapes=[
            pltpu.VMEM((BL, G3), jnp.bfloat16),
            pltpu.VMEM((BL, G3), jnp.bfloat16),
            pltpu.VMEM((BL, 2 * HP), jnp.bfloat16),
            pltpu.VMEM((BL, 2 * HP), jnp.bfloat16),
        ],
    )(x, *args)


def kernel(dataTokens, embed,
           Wih0f, Whh0f, bih0f, bhh0f, Wih0b, Whh0b, bih0b, bhh0b,
           Wih1f, Whh1f, bih1f, bhh1f, Wih1b, Whh1b, bih1b, bhh1b,
           Wih2f, Whh2f, bih2f, bhh2f, Wih2b, Whh2b, bih2b, bhh2b,
           W1, b1, W11, b11, W2, b2):
    bft = jnp.bfloat16
    idx = dataTokens.T.reshape(-1).astype(jnp.int32)
    idx = jnp.pad(idx, (0, BL_PAD - BL))
    table = jnp.pad(embed, ((0, 0), (0, EP - E)))
    rows = _sc_gather(table, idx)

    wih0 = _gate_pad_rows(jnp.stack([Wih0f, Wih0b]))
    wih0 = jnp.pad(wih0, ((0, 0), (0, 0), (0, EP - E))).astype(bft)
    wih12 = _in_remap(_gate_pad_rows(
        jnp.stack([Wih1f, Wih1b, Wih2f, Wih2b]))).astype(bft)
    whh = _gate_pad_rows(jnp.stack(
        [Whh0f, Whh0b, Whh1f, Whh1b, Whh2f, Whh2b]))
    whh = jnp.pad(jnp.swapaxes(whh, 1, 2),
                  ((0, 0), (0, HP - H), (0, 0))).astype(bft)
    bih = jnp.pad(jnp.stack([bih0f, bih0b, bih1f, bih1b, bih2f, bih2b])
                  .reshape(6, 3, H), ((0, 0), (0, 0), (0, HP - H))).reshape(6, G3)
    bhh = jnp.pad(jnp.stack([bhh0f, bhh0b, bhh1f, bhh1b, bhh2f, bhh2b])
                  .reshape(6, 3, H), ((0, 0), (0, 0), (0, HP - H))).reshape(6, G3)

    w2p = jnp.pad(W2, ((0, 128 - 4), (0, 0)))
    b2p = jnp.pad(b2, (0, 128 - 4)).reshape(1, 128)

    out = _tc_forward(rows, (wih0, wih12, whh, bih, bhh, W1, W1,
                             b1.reshape(1, 1000), W11, b11.reshape(1, 500),
                             w2p, b2p))
    return out[:, 0:4]


# speedup vs baseline: 1.2076x; 1.0011x over previous
import functools

import jax
import jax.numpy as jnp
from jax import lax
from jax.experimental import pallas as pl
from jax.experimental.pallas import tpu as pltpu
from jax.experimental.pallas import tpu_sc as plsc

B = 64
L = 50
BL = B * L
H = 200
HP = 256
G3 = 3 * HP
E = 100
EP = 128
BL_PAD = 3328


def _sc_gather(table, idx):
    info = plsc.get_sparse_core_info()
    nc, ns = info.num_cores, info.num_subcores
    nw = nc * ns
    b_per_w = BL_PAD // nw

    mesh = plsc.VectorSubcoreMesh(core_axis_name="c", subcore_axis_name="s")

    @functools.partial(
        pl.kernel,
        mesh=mesh,
        out_type=jax.ShapeDtypeStruct((BL_PAD, EP), jnp.float32),
        scratch_types=[
            pltpu.VMEM((b_per_w,), jnp.int32),
            pltpu.VMEM((b_per_w, EP), jnp.float32),
            pltpu.SemaphoreType.DMA,
        ],
    )
    def gather_kernel(table_hbm, idx_hbm, out_hbm, idx_v, rows_v, sem):
        wid = lax.axis_index("s") * nc + lax.axis_index("c")
        base = wid * b_per_w
        pltpu.sync_copy(idx_hbm.at[pl.ds(base, b_per_w)], idx_v)
        pltpu.async_copy(table_hbm.at[idx_v], rows_v, sem).wait()
        pltpu.sync_copy(rows_v, out_hbm.at[pl.ds(base, b_per_w)])

    return gather_kernel(table, idx)


def _gate_pad_rows(W):
    lead = W.shape[:-2]
    cin = W.shape[-1]
    Wg = W.reshape(lead + (3, H, cin))
    pad = [(0, 0)] * len(lead) + [(0, 0), (0, HP - H), (0, 0)]
    return jnp.pad(Wg, pad).reshape(lead + (G3, cin))


def _in_remap(W):
    z = jnp.zeros(W.shape[:-1] + (HP - H,), W.dtype)
    return jnp.concatenate([W[..., 0:H], z, W[..., H:2 * H], z], axis=-1)


def _nt(a, b):
    return lax.dot_general(a, b, (((1,), (1,)), ((), ())),
                           preferred_element_type=jnp.float32)


def _tc_body(x_ref, wih0, wih12, whh, bih, bhh,
             w1a, w1b, b1, w11, b11, w2p, b2,
             out_ref, gf, gb, y1, y2):
    xin = (x_ref, y1, y2)
    yout = (y1, y2, None)

    finals = []
    for l in range(3):
        wf = wih0[0] if l == 0 else wih12[2 * l - 2]
        wb = wih0[1] if l == 0 else wih12[2 * l - 1]
        bf = bih[2 * l:2 * l + 1, :]
        bb = bih[2 * l + 1:2 * l + 2, :]
        nch = 4
        rows = BL // nch
        for c in range(nch):
            if l == 0:
                xs = x_ref[c * rows:(c + 1) * rows, :].astype(jnp.bfloat16)
            else:
                xs = xin[l][c * rows:(c + 1) * rows, :]
            gf[c * rows:(c + 1) * rows, :] = _nt(xs, wf) + bf
            gb[c * rows:(c + 1) * rows, :] = _nt(xs, wb) + bb

        whf = whh[2 * l]
        whb = whh[2 * l + 1]
        bhf = bhh[2 * l:2 * l + 1, :]
        bhb = bhh[2 * l + 1:2 * l + 2, :]
        ydst = yout[l]

        def step(t, h, whf=whf, whb=whb, bhf=bhf, bhb=bhb, ydst=ydst):
            gif = gf[pl.ds(t * B, B), :]
            gib = gb[pl.ds((L - 1) * B - t * B, B), :]
            gi = jnp.concatenate([gif, gib], axis=0)
            hb16 = h.astype(jnp.bfloat16)
            ghf = lax.dot(hb16[0:B], whf, preferred_element_type=jnp.float32) + bhf
            ghb = lax.dot(hb16[B:2 * B], whb, preferred_element_type=jnp.float32) + bhb
            gh = jnp.concatenate([ghf, ghb], axis=0)
            rz = jax.nn.sigmoid(gi[:, 0:2 * HP] + gh[:, 0:2 * HP])
            r = rz[:, 0:HP]
            z = rz[:, HP:2 * HP]
            n = jnp.tanh(gi[:, 2 * HP:G3] + r * gh[:, 2 * HP:G3])
            hn = (1.0 - z) * n + z * h
            if ydst is not None:
                hnb = hn.astype(jnp.bfloat16)
                ydst[pl.ds(t * B, B), 0:HP] = hnb[0:B]
                ydst[pl.ds((L - 1) * B - t * B, B), HP:2 * HP] = hnb[B:2 * B]
            return hn

        h = lax.fori_loop(0, L, step, jnp.zeros((2 * B, HP), jnp.float32))
        finals.append(h[0:B])
        finals.append(h[B:2 * B])

    x1c = jnp.concatenate([f[:, 0:H] for f in finals], axis=1)
    h1 = jnp.maximum(_nt(x1c[:, 0:560], w1a[:, 80:640])
                     + _nt(x1c[:, 560:1200], w1b[...]) + b1[...], 0.0)
    h2 = jnp.maximum(_nt(h1, w11[...]) + b11[...], 0.0)
    out_ref[...] = jnp.maximum(_nt(h2, w2p[...]) + b2[...], 0.0)


def _tc_forward(x, args):
    full = pl.BlockSpec()
    in_specs = [full] * 6 + [pl.BlockSpec((1000, 640), lambda i: (0, 3)),
                             pl.BlockSpec((1000, 640), lambda i: (0, 4))] + [full] * 5
    return pl.pallas_call(
        _tc_body,
        out_shape=jax.ShapeDtypeStruct((B, 128), jnp.float32),
        grid=(1,),
        in_specs=in_specs,
        scratch_shapes=[
            pltpu.VMEM((BL, G3), jnp.float32),
            pltpu.VMEM((BL, G3), jnp.float32),
            pltpu.VMEM((BL, 2 * HP), jnp.bfloat16),
            pltpu.VMEM((BL, 2 * HP), jnp.bfloat16),
        ],
    )(x, *args)


def kernel(dataTokens, embed,
           Wih0f, Whh0f, bih0f, bhh0f, Wih0b, Whh0b, bih0b, bhh0b,
           Wih1f, Whh1f, bih1f, bhh1f, Wih1b, Whh1b, bih1b, bhh1b,
           Wih2f, Whh2f, bih2f, bhh2f, Wih2b, Whh2b, bih2b, bhh2b,
           W1, b1, W11, b11, W2, b2):
    bft = jnp.bfloat16
    idx = dataTokens.T.reshape(-1).astype(jnp.int32)
    idx = jnp.pad(idx, (0, BL_PAD - BL))
    table = jnp.pad(embed, ((0, 0), (0, EP - E)))
    rows = _sc_gather(table, idx)

    wih0 = _gate_pad_rows(jnp.stack([Wih0f, Wih0b]))
    wih0 = jnp.pad(wih0, ((0, 0), (0, 0), (0, EP - E))).astype(bft)
    wih12 = _in_remap(_gate_pad_rows(
        jnp.stack([Wih1f, Wih1b, Wih2f, Wih2b]))).astype(bft)
    whh = _gate_pad_rows(jnp.stack(
        [Whh0f, Whh0b, Whh1f, Whh1b, Whh2f, Whh2b]))
    whh = jnp.pad(jnp.swapaxes(whh, 1, 2),
                  ((0, 0), (0, HP - H), (0, 0))).astype(bft)
    bih = jnp.pad(jnp.stack([bih0f, bih0b, bih1f, bih1b, bih2f, bih2b])
                  .reshape(6, 3, H), ((0, 0), (0, 0), (0, HP - H))).reshape(6, G3)
    bhh = jnp.pad(jnp.stack([bhh0f, bhh0b, bhh1f, bhh1b, bhh2f, bhh2b])
                  .reshape(6, 3, H), ((0, 0), (0, 0), (0, HP - H))).reshape(6, G3)

    w2p = jnp.pad(W2, ((0, 128 - 4), (0, 0)))
    b2p = jnp.pad(b2, (0, 128 - 4)).reshape(1, 128)

    out = _tc_forward(rows, (wih0, wih12, whh, bih, bhh, W1, W1,
                             b1.reshape(1, 1000), W11, b11.reshape(1, 500),
                             w2p, b2p))
    return out[:, 0:4]
